# Initial kernel scaffold; baseline (speedup 1.0000x reference)
import jax
import jax.numpy as jnp
from jax.experimental import pallas as pl


def kernel(inp, W):
    raise NotImplementedError("write your pallas kernel here")


# speedup vs baseline: 5.1676x; 5.1676x over previous
import functools

import jax
import jax.numpy as jnp
from jax import lax
from jax.experimental import pallas as pl
from jax.experimental.pallas import tpu as pltpu
from jax.experimental.pallas import tpu_sc as plsc

_NC = 2
_NS = 16
_NW = _NC * _NS
_CHUNK = 128


def _scale_table(W, scale):
    V, D = W.shape
    scale = float(scale)
    rb = next(r for r in (2000, 1000, 500, 250, 200, 100, 50, 25, 8, 1)
              if V % r == 0)

    def body(w_ref, o_ref):
        o_ref[...] = w_ref[...] * scale

    return pl.pallas_call(
        body,
        out_shape=jax.ShapeDtypeStruct((V, D), W.dtype),
        grid=(V // rb,),
        in_specs=[pl.BlockSpec((rb, D), lambda i: (i, 0))],
        out_specs=pl.BlockSpec((rb, D), lambda i: (i, 0)),
    )(W)


@functools.partial(jax.jit, static_argnums=())
def _sc_gather(table, idx):
    (B,) = idx.shape
    V, D = table.shape
    assert B % (_NW * _CHUNK) == 0
    b_per_w = B // _NW
    n_chunk = b_per_w // _CHUNK
    mesh = plsc.VectorSubcoreMesh(core_axis_name="c", subcore_axis_name="s")

    @functools.partial(
        pl.kernel,
        mesh=mesh,
        out_type=jax.ShapeDtypeStruct((B, D), table.dtype),
        scratch_types=[
            pltpu.VMEM((b_per_w,), jnp.int32),
            pltpu.VMEM((2, _CHUNK, D), table.dtype),
            pltpu.SemaphoreType.DMA,
            pltpu.SemaphoreType.DMA,
        ],
    )
    def k(table_hbm, idx_hbm, out_hbm, idx_v, rows_v, sem0, sem1):
        wid = lax.axis_index("s") * _NC + lax.axis_index("c")
        base = wid * b_per_w
        pltpu.sync_copy(idx_hbm.at[pl.ds(base, b_per_w)], idx_v)
        sems = (sem0, sem1)

        def gather(i, buf):
            pltpu.async_copy(
                table_hbm.at[idx_v.at[pl.ds(i * _CHUNK, _CHUNK)]],
                rows_v.at[buf], sems[buf])

        def wait_gather(buf):
            pltpu.make_async_copy(
                table_hbm.at[idx_v.at[pl.ds(0, _CHUNK)]],
                rows_v.at[buf], sems[buf]).wait()

        gather(0, 0)
        if n_chunk > 1:
            gather(1, 1)

        @pl.loop(0, n_chunk, step=2)
        def _(g):
            for b in range(2):
                i = g + b
                wait_gather(b)
                pltpu.sync_copy(
                    rows_v.at[b],
                    out_hbm.at[pl.ds(base + i * _CHUNK, _CHUNK)])
                @pl.when(i + 2 < n_chunk)
                def _():
                    gather(i + 2, b)

    return k(table, idx)


def kernel(inp, W):
    B0, S = inp.shape
    V, D = W.shape
    Ws = _scale_table(W, float(D) ** 0.5)
    idx = inp.reshape(B0 * S).astype(jnp.int32)
    out = _sc_gather(Ws, idx)
    return out.reshape(B0, S, D)


# speedup vs baseline: 7.8470x; 1.5185x over previous
import functools

import jax
import jax.numpy as jnp
from jax import lax
from jax.experimental import pallas as pl
from jax.experimental.pallas import tpu as pltpu
from jax.experimental.pallas import tpu_sc as plsc

_NC = 2
_NS = 16
_NW = _NC * _NS
_CHUNK = 128
_L = 16


def _sc_gather_scale(table, idx, scale):
    (B,) = idx.shape
    V, D = table.shape
    assert B % (_NW * _CHUNK) == 0 and D % _L == 0
    b_per_w = B // _NW
    n_chunk = b_per_w // _CHUNK
    scale = float(scale)
    mesh = plsc.VectorSubcoreMesh(core_axis_name="c", subcore_axis_name="s")

    @functools.partial(
        pl.kernel,
        mesh=mesh,
        out_type=jax.ShapeDtypeStruct((B, D), table.dtype),
        scratch_types=[
            pltpu.VMEM((b_per_w,), jnp.int32),
            pltpu.VMEM((2, _CHUNK, D), table.dtype),
            pltpu.VMEM((2, _CHUNK, D), table.dtype),
            pltpu.SemaphoreType.DMA,
            pltpu.SemaphoreType.DMA,
            pltpu.SemaphoreType.DMA,
            pltpu.SemaphoreType.DMA,
        ],
    )
    def k(table_hbm, idx_hbm, out_hbm, idx_v, gbuf, sbuf,
          gsem0, gsem1, ssem0, ssem1):
        wid = lax.axis_index("s") * _NC + lax.axis_index("c")
        base = wid * b_per_w
        pltpu.sync_copy(idx_hbm.at[pl.ds(base, b_per_w)], idx_v)
        gsems = (gsem0, gsem1)
        ssems = (ssem0, ssem1)

        def gather(i, b):
            pltpu.async_copy(
                table_hbm.at[idx_v.at[pl.ds(i * _CHUNK, _CHUNK)]],
                gbuf.at[b], gsems[b])

        def drain(ref, sem):
            pltpu.make_async_copy(
                table_hbm.at[idx_v.at[pl.ds(0, _CHUNK)]], ref, sem).wait()

        gather(0, 0)
        gather(1, 1)

        @pl.loop(0, n_chunk, step=2)
        def _(g):
            for b in range(2):
                i = g + b
                drain(gbuf.at[b], gsems[b])

                @pl.when(i >= 2)
                def _():
                    drain(sbuf.at[b], ssems[b])

                @pl.loop(0, _CHUNK)
                def _(r):
                    for j in range(D // _L):
                        sl = pl.ds(j * _L, _L)
                        sbuf[b, r, sl] = gbuf[b, r, sl] * scale

                @pl.when(i + 2 < n_chunk)
                def _():
                    gather(i + 2, b)
                pltpu.async_copy(
                    sbuf.at[b],
                    out_hbm.at[pl.ds(base + i * _CHUNK, _CHUNK)], ssems[b])

        drain(sbuf.at[0], ssems[0])
        drain(sbuf.at[1], ssems[1])

    return k(table, idx)


def kernel(inp, W):
    B0, S = inp.shape
    V, D = W.shape
    idx = inp.reshape(B0 * S).astype(jnp.int32)
    out = _sc_gather_scale(W, idx, float(D) ** 0.5)
    return out.reshape(B0, S, D)
